# Initial kernel scaffold; baseline (speedup 1.0000x reference)
import jax
import jax.numpy as jnp
from jax.experimental import pallas as pl


def kernel(inputs):
    raise NotImplementedError("write your pallas kernel here")


# speedup vs baseline: 1.3202x; 1.3202x over previous
import jax
import jax.numpy as jnp
from jax.experimental import pallas as pl

_K = 64
_N = 2048
_B = 128


def _topk_block(x_ref, o_ref):
    x0 = x_ref[...]
    iota = jax.lax.broadcasted_iota(jnp.int32, x0.shape, 1)
    iota_k = jax.lax.broadcasted_iota(jnp.int32, (x0.shape[0], _K), 1)
    out0 = jnp.zeros((x0.shape[0], _K), jnp.float32)

    def body(t, carry):
        x, out = carry
        m = jnp.max(x, axis=1)
        eq = x == m[:, None]
        cand = jnp.where(eq, iota, _N)
        i0 = jnp.min(cand, axis=1)
        out = jnp.where(iota_k == t, m[:, None], out)
        x = jnp.where(iota == i0[:, None], -jnp.inf, x)
        return x, out

    _, out = jax.lax.fori_loop(0, _K, body, (x0, out0))
    o_ref[...] = out


def kernel(inputs):
    b, s, n = inputs.shape
    rows = b * s
    x = inputs.reshape(rows, n)
    out = pl.pallas_call(
        _topk_block,
        grid=(rows // _B,),
        in_specs=[pl.BlockSpec((_B, n), lambda i: (i, 0))],
        out_specs=pl.BlockSpec((_B, _K), lambda i: (i, 0)),
        out_shape=jax.ShapeDtypeStruct((rows, _K), jnp.float32),
    )(x)
    return out.reshape(b, s, _K)


# speedup vs baseline: 6.1335x; 4.6459x over previous
import functools

import jax
import jax.numpy as jnp
from jax import lax
from jax.experimental import pallas as pl
from jax.experimental.pallas import tpu as pltpu
from jax.experimental.pallas import tpu_sc as plsc

_K = 64
_N = 2048
_L = 16
_NVEC = _N // _L
_NGRP = 16
_NC = 2
_NS = 16
_NW = _NC * _NS
_CHUNK = 32


def _sortd(v):
    k, _ = plsc.sort_key_val(v, v, descending=True)
    return k


def _merge_insert(a, b):
    out = []
    for i in range(4):
        rb = lax.rev(b, (0,))
        hi = jnp.maximum(a[i], rb)
        lo = jnp.minimum(a[i], rb)
        out.append(_sortd(hi))
        b = _sortd(lo)
    return tuple(out)


def _lane_4th_largest(ms):
    mx, mn = jnp.maximum, jnp.minimum
    pairs = [(mx(ms[2 * i], ms[2 * i + 1]), mn(ms[2 * i], ms[2 * i + 1]))
             for i in range(8)]

    def merge22(a, b):
        c0 = mx(a[0], b[0])
        c3 = mn(a[1], b[1])
        t1 = mn(a[0], b[0])
        t2 = mx(a[1], b[1])
        return (c0, mx(t1, t2), mn(t1, t2), c3)

    quads = [merge22(pairs[2 * i], pairs[2 * i + 1]) for i in range(4)]

    def merge44_top4(a, b):
        h = [mx(a[i], b[3 - i]) for i in range(4)]
        d0, d2 = mx(h[0], h[2]), mn(h[0], h[2])
        d1, d3 = mx(h[1], h[3]), mn(h[1], h[3])
        return (mx(d0, d1), mn(d0, d1), mx(d2, d3), mn(d2, d3))

    t_a = merge44_top4(quads[0], quads[1])
    t_b = merge44_top4(quads[2], quads[3])
    h = [mx(t_a[i], t_b[3 - i]) for i in range(4)]
    return mn(mn(h[0], h[1]), mn(h[2], h[3]))


def _sc_body(rows_w, x_hbm, out_hbm, xbuf, cand, mxbuf, obuf):
    wid = lax.axis_index("s") * _NC + lax.axis_index("c")
    base_row = wid * rows_w
    neg = jnp.full((_L,), -jnp.inf, jnp.float32)

    def chunk_body(ch, _):
        row0 = base_row + ch * _CHUNK
        pltpu.sync_copy(x_hbm.at[pl.ds(row0 * _N, _CHUNK * _N)], xbuf)

        def row_body(r, _):
            roff = r * _N

            def grp_body(g, _):
                off = roff + g * (_L * 8)
                m = xbuf[pl.ds(off, _L)]
                for j in range(1, 8):
                    m = jnp.maximum(m, xbuf[pl.ds(off + j * _L, _L)])
                mxbuf[pl.ds(g * _L, _L)] = m
                return 0

            lax.fori_loop(0, _NGRP, grp_body, 0)

            ms = [mxbuf[pl.ds(g * _L, _L)] for g in range(_NGRP)]
            t_thresh = jnp.min(_lane_4th_largest(ms))
            tv = jnp.full((_L,), t_thresh, jnp.float32)

            def cand_body(i, c):
                v = xbuf[pl.ds(roff + i * _L, _L)]
                mask = v >= tv
                plsc.store_compressed(cand.at[pl.ds(c, _L)], v, mask=mask)
                return c + jnp.sum(mask.astype(jnp.int32))

            c = lax.fori_loop(0, _NVEC, cand_body, jnp.int32(0))
            cand[pl.ds(c, _L)] = neg

            nv = (c + _L - 1) // _L

            def merge_body(j, a):
                b = _sortd(cand[pl.ds(j * _L, _L)])
                return _merge_insert(a, b)

            a = lax.fori_loop(0, nv, merge_body, (neg, neg, neg, neg))
            for i in range(4):
                obuf[pl.ds(r * _K + i * _L, _L)] = a[i]
            return 0

        lax.fori_loop(0, _CHUNK, row_body, 0)
        pltpu.sync_copy(obuf, out_hbm.at[pl.ds(row0 * _K, _CHUNK * _K)])
        return 0

    lax.fori_loop(0, rows_w // _CHUNK, chunk_body, 0)


def kernel(inputs):
    b, s, n = inputs.shape
    rows = b * s
    x = inputs.reshape(rows * n)
    mesh = plsc.VectorSubcoreMesh(
        core_axis_name="c", subcore_axis_name="s",
        num_cores=_NC, num_subcores=_NS)
    f = pl.kernel(
        functools.partial(_sc_body, rows // _NW),
        out_type=jax.ShapeDtypeStruct((rows * _K,), jnp.float32),
        mesh=mesh,
        compiler_params=pltpu.CompilerParams(needs_layout_passes=False),
        scratch_types=[
            pltpu.VMEM((_CHUNK * _N,), jnp.float32),
            pltpu.VMEM((_N + _L,), jnp.float32),
            pltpu.VMEM((_NGRP * _L,), jnp.float32),
            pltpu.VMEM((_CHUNK * _K,), jnp.float32),
        ],
    )
    return f(x).reshape(b, s, _K)
